# baseline (speedup 1.0000x reference)
import functools

import jax
import jax.numpy as jnp
from jax import lax
from jax.experimental import pallas as pl
from jax.experimental.pallas import tpu as pltpu
from jax.experimental.pallas import tpu_sc as plsc

SEQ = 200
BATCH = 4096
EMBED = 128
NC = 2
NS = 16
NW = NC * NS
BPW = BATCH // NW
HPW = BPW // 2
LANES = 16
NCH = EMBED // LANES
NQ = SEQ // 4
UNROLL = 4
INV_SEQ = 1.0 / SEQ


def _w2v_body(sent, table, out, ih0, ih1, b0, b1, b2, b3, tacc,
              s0, s1, s2, s3):
    c = lax.axis_index("c")
    s = lax.axis_index("s")
    wid = s * NC + c
    base = wid * BPW
    idxh = (ih0, ih1)
    bufs = (b0, b1, b2, b3)
    sems = (s0, s1, s2, s3)

    pltpu.sync_copy(sent.at[:, 2 * wid], ih0)
    pltpu.sync_copy(sent.at[:, 2 * wid + 1], ih1)

    def gather_quad(q, h):
        for i in range(2):
            pltpu.async_copy(
                table.at[idxh[h].at[2 * q + i]],
                bufs[2 * h + i], sems[2 * h + i])

    def wait_quad(q, h):
        for i in range(2):
            pltpu.make_async_copy(
                table.at[idxh[h].at[2 * q + i]],
                bufs[2 * h + i], sems[2 * h + i]).wait()

    def accumulate_quad(h, init):
        ba, bb = bufs[2 * h], bufs[2 * h + 1]
        hr = h * HPW

        def abody(r4, carry):
            for ur in range(UNROLL):
                r = UNROLL * r4 + ur
                for ch in range(NCH):
                    sl = pl.ds(ch * LANES, LANES)
                    v = (ba[r, sl] + ba[HPW + r, sl]) + \
                        (bb[r, sl] + bb[HPW + r, sl])
                    if init:
                        tacc[hr + r, sl] = v
                    else:
                        plsc.addupdate(tacc.at[hr + r, sl], v)
            return carry

        lax.fori_loop(0, HPW // UNROLL, abody, 0)

    gather_quad(0, 0)
    gather_quad(0, 1)

    wait_quad(0, 0)
    accumulate_quad(0, init=True)
    gather_quad(1, 0)
    wait_quad(0, 1)
    accumulate_quad(1, init=True)
    gather_quad(1, 1)

    def gbody(g, carry):
        q = g + 1
        wait_quad(q, 0)
        accumulate_quad(0, init=False)
        gather_quad(q + 1, 0)
        wait_quad(q, 1)
        accumulate_quad(1, init=False)
        gather_quad(q + 1, 1)
        return carry

    lax.fori_loop(0, NQ - 2, gbody, 0)

    wait_quad(NQ - 1, 0)
    accumulate_quad(0, init=False)
    wait_quad(NQ - 1, 1)
    accumulate_quad(1, init=False)

    def sbody(r, carry):
        for ch in range(NCH):
            sl = pl.ds(ch * LANES, LANES)
            tacc[r, sl] = tacc[r, sl] * INV_SEQ
        return carry

    lax.fori_loop(0, BPW, sbody, 0)
    pltpu.sync_copy(tacc, out.at[pl.ds(base, BPW)])


@jax.jit
def kernel(sentence, table):
    sentence = (
        sentence.astype(jnp.int32)
        .reshape(SEQ // 2, 2, BATCH // HPW, HPW)
        .transpose(0, 2, 1, 3)
        .reshape(SEQ // 2, BATCH // HPW, 2 * HPW)
    )
    mesh = plsc.VectorSubcoreMesh(
        core_axis_name="c", subcore_axis_name="s", num_cores=NC, num_subcores=NS
    )
    k = functools.partial(
        pl.kernel,
        out_type=jax.ShapeDtypeStruct((BATCH, EMBED), jnp.float32),
        mesh=mesh,
        scratch_types=(
            [pltpu.VMEM((SEQ // 2, BPW), jnp.int32)] * 2
            + [pltpu.VMEM((BPW, EMBED), jnp.float32)] * 4
            + [pltpu.VMEM((BPW, EMBED), jnp.float32)]
            + [pltpu.SemaphoreType.DMA] * 4
        ),
    )(_w2v_body)
    return k(sentence, table)


# speedup vs baseline: 1.1098x; 1.1098x over previous
import functools

import jax
import jax.numpy as jnp
from jax import lax
from jax.experimental import pallas as pl
from jax.experimental.pallas import tpu as pltpu
from jax.experimental.pallas import tpu_sc as plsc

SEQ = 200
BATCH = 4096
EMBED = 128
NC = 2
NS = 16
NW = NC * NS
BPW = BATCH // NW
HPW = BPW // 2
LANES = 16
NCH = EMBED // LANES
NT = 66
UNROLL = 4
INV_SEQ = 1.0 / SEQ


def _w2v_body(sent, table, out, idx_v, b0, b1, b2, b3, b4, b5, b6, b7, b8,
              tacc, s0, s1, s2, s3, s4, s5, s6, s7, s8):
    c = lax.axis_index("c")
    s = lax.axis_index("s")
    wid = s * NC + c
    base = wid * BPW
    bufs = (b0, b1, b2, b3, b4, b5, b6, b7, b8)
    sems = (s0, s1, s2, s3, s4, s5, s6, s7, s8)

    pltpu.sync_copy(sent.at[:, pl.ds(base, BPW)], idx_v)

    def gather_one(l, h, k):
        pltpu.async_copy(
            table.at[idx_v.at[l, pl.ds(h * HPW, HPW)]], bufs[k], sems[k])

    def gather_triple(t, h, g):
        for i in range(3):
            gather_one(3 * t + i, h, 3 * g + i)

    def wait_triple(t, h, g):
        for i in range(3):
            pltpu.make_async_copy(
                table.at[idx_v.at[3 * t + i, pl.ds(h * HPW, HPW)]],
                bufs[3 * g + i], sems[3 * g + i]).wait()

    def accumulate_triple(g, h, init):
        ba, bb, bc = bufs[3 * g], bufs[3 * g + 1], bufs[3 * g + 2]
        hr = h * HPW

        def abody(r4, carry):
            for ur in range(UNROLL):
                r = UNROLL * r4 + ur
                for ch in range(NCH):
                    sl = pl.ds(ch * LANES, LANES)
                    v = (ba[r, sl] + bb[r, sl]) + bc[r, sl]
                    if init:
                        tacc[hr + r, sl] = v
                    else:
                        plsc.addupdate(tacc.at[hr + r, sl], v)
            return carry

        lax.fori_loop(0, HPW // UNROLL, abody, 0)

    def unit(g, h, t, init=False, refill=None):
        wait_triple(t, h, g)
        accumulate_triple(g, h, init)
        if refill is not None:
            rh, rt = refill
            gather_triple(rt, rh, g)

    gather_triple(0, 0, 0)
    gather_triple(0, 1, 1)
    gather_triple(1, 0, 2)

    unit(0, 0, 0, init=True, refill=(1, 1))
    unit(1, 1, 0, init=True, refill=(0, 2))
    unit(2, 0, 1, refill=(1, 2))

    def gbody(sp, carry):
        t3 = 3 * sp
        unit(0, 1, t3 + 1, refill=(0, t3 + 3))
        unit(1, 0, t3 + 2, refill=(1, t3 + 3))
        unit(2, 1, t3 + 2, refill=(0, t3 + 4))
        unit(0, 0, t3 + 3, refill=(1, t3 + 4))
        unit(1, 1, t3 + 3, refill=(0, t3 + 5))
        unit(2, 0, t3 + 4, refill=(1, t3 + 5))
        return carry

    lax.fori_loop(0, 21, gbody, 0)

    unit(0, 1, NT - 2)
    gather_one(SEQ - 2, 0, 0)
    gather_one(SEQ - 1, 0, 1)
    unit(1, 0, NT - 1)
    gather_one(SEQ - 2, 1, 3)
    gather_one(SEQ - 1, 1, 4)
    unit(2, 1, NT - 1)

    def accumulate_pair(ka, kb, h):
        ba, bb = bufs[ka], bufs[kb]
        hr = h * HPW

        def pbody(r4, carry):
            for ur in range(UNROLL):
                r = UNROLL * r4 + ur
                for ch in range(NCH):
                    sl = pl.ds(ch * LANES, LANES)
                    plsc.addupdate(tacc.at[hr + r, sl], ba[r, sl] + bb[r, sl])
            return carry

        lax.fori_loop(0, HPW // UNROLL, pbody, 0)

    for k, (l, h) in ((0, (SEQ - 2, 0)), (1, (SEQ - 1, 0)),
                      (3, (SEQ - 2, 1)), (4, (SEQ - 1, 1))):
        pltpu.make_async_copy(
            table.at[idx_v.at[l, pl.ds(h * HPW, HPW)]],
            bufs[k], sems[k]).wait()
    accumulate_pair(0, 1, 0)
    accumulate_pair(3, 4, 1)

    def sbody(r, carry):
        for ch in range(NCH):
            sl = pl.ds(ch * LANES, LANES)
            tacc[r, sl] = tacc[r, sl] * INV_SEQ
        return carry

    lax.fori_loop(0, BPW, sbody, 0)
    pltpu.sync_copy(tacc, out.at[pl.ds(base, BPW)])


@jax.jit
def kernel(sentence, table):
    sentence = sentence.astype(jnp.int32)
    mesh = plsc.VectorSubcoreMesh(
        core_axis_name="c", subcore_axis_name="s", num_cores=NC, num_subcores=NS
    )
    k = functools.partial(
        pl.kernel,
        out_type=jax.ShapeDtypeStruct((BATCH, EMBED), jnp.float32),
        mesh=mesh,
        scratch_types=(
            [pltpu.VMEM((SEQ, BPW), jnp.int32)]
            + [pltpu.VMEM((HPW, EMBED), jnp.float32)] * 9
            + [pltpu.VMEM((BPW, EMBED), jnp.float32)]
            + [pltpu.SemaphoreType.DMA] * 9
        ),
    )(_w2v_body)
    return k(sentence, table)


# speedup vs baseline: 1.1150x; 1.0047x over previous
import functools

import jax
import jax.numpy as jnp
from jax import lax
from jax.experimental import pallas as pl
from jax.experimental.pallas import tpu as pltpu
from jax.experimental.pallas import tpu_sc as plsc

SEQ = 200
BATCH = 4096
EMBED = 128
NC = 2
NS = 16
NW = NC * NS
BPW = BATCH // NW
HPW = BPW // 2
LANES = 16
NCH = EMBED // LANES
NT = 66
UNROLL = 4
INV_SEQ = 1.0 / SEQ


def _w2v_body(sent, table, out, idx_v, g0, g1, g2, tacc, s0, s1, s2):
    c = lax.axis_index("c")
    s = lax.axis_index("s")
    wid = s * NC + c
    base = wid * BPW
    grps = (g0, g1, g2)
    sems = (s0, s1, s2)

    pltpu.sync_copy(sent.at[:, pl.ds(base, BPW)], idx_v)

    def gather_triple(t, h, g):
        for i in range(3):
            pltpu.async_copy(
                table.at[idx_v.at[3 * t + i, pl.ds(h * HPW, HPW)]],
                grps[g].at[pl.ds(i * HPW, HPW)], sems[g])

    def wait_group(g):
        pltpu.make_async_copy(
            table.at[pl.ds(0, 3 * HPW)], grps[g], sems[g]).wait()

    def accumulate_triple(g, h, init):
        gb = grps[g]
        hr = h * HPW

        def abody(r4, carry):
            for ur in range(UNROLL):
                r = UNROLL * r4 + ur
                for ch in range(NCH):
                    sl = pl.ds(ch * LANES, LANES)
                    v = (gb[r, sl] + gb[HPW + r, sl]) + gb[2 * HPW + r, sl]
                    if init:
                        tacc[hr + r, sl] = v
                    else:
                        plsc.addupdate(tacc.at[hr + r, sl], v)
            return carry

        lax.fori_loop(0, HPW // UNROLL, abody, 0)

    def unit(g, h, t, init=False, refill=None):
        wait_group(g)
        accumulate_triple(g, h, init)
        if refill is not None:
            rh, rt = refill
            gather_triple(rt, rh, g)

    gather_triple(0, 0, 0)
    gather_triple(0, 1, 1)
    gather_triple(1, 0, 2)

    unit(0, 0, 0, init=True, refill=(1, 1))
    unit(1, 1, 0, init=True, refill=(0, 2))
    unit(2, 0, 1, refill=(1, 2))

    def gbody(sp, carry):
        t3 = 3 * sp
        unit(0, 1, t3 + 1, refill=(0, t3 + 3))
        unit(1, 0, t3 + 2, refill=(1, t3 + 3))
        unit(2, 1, t3 + 2, refill=(0, t3 + 4))
        unit(0, 0, t3 + 3, refill=(1, t3 + 4))
        unit(1, 1, t3 + 3, refill=(0, t3 + 5))
        unit(2, 0, t3 + 4, refill=(1, t3 + 5))
        return carry

    lax.fori_loop(0, 21, gbody, 0)

    def gather_pair(h, g):
        for i in range(2):
            pltpu.async_copy(
                table.at[idx_v.at[SEQ - 2 + i, pl.ds(h * HPW, HPW)]],
                grps[g].at[pl.ds(i * HPW, HPW)], sems[g])

    unit(0, 1, NT - 2)
    gather_pair(0, 0)
    unit(1, 0, NT - 1)
    gather_pair(1, 1)
    unit(2, 1, NT - 1)

    def accumulate_pair(g, h):
        gb = grps[g]
        hr = h * HPW

        def pbody(r4, carry):
            for ur in range(UNROLL):
                r = UNROLL * r4 + ur
                for ch in range(NCH):
                    sl = pl.ds(ch * LANES, LANES)
                    plsc.addupdate(tacc.at[hr + r, sl],
                                   gb[r, sl] + gb[HPW + r, sl])
            return carry

        lax.fori_loop(0, HPW // UNROLL, pbody, 0)

    for g in range(2):
        pltpu.make_async_copy(
            table.at[pl.ds(0, 2 * HPW)],
            grps[g].at[pl.ds(0, 2 * HPW)], sems[g]).wait()
    accumulate_pair(0, 0)
    accumulate_pair(1, 1)

    def sbody(r, carry):
        for ch in range(NCH):
            sl = pl.ds(ch * LANES, LANES)
            tacc[r, sl] = tacc[r, sl] * INV_SEQ
        return carry

    lax.fori_loop(0, BPW, sbody, 0)
    pltpu.sync_copy(tacc, out.at[pl.ds(base, BPW)])


@jax.jit
def kernel(sentence, table):
    sentence = sentence.astype(jnp.int32)
    mesh = plsc.VectorSubcoreMesh(
        core_axis_name="c", subcore_axis_name="s", num_cores=NC, num_subcores=NS
    )
    k = functools.partial(
        pl.kernel,
        out_type=jax.ShapeDtypeStruct((BATCH, EMBED), jnp.float32),
        mesh=mesh,
        scratch_types=(
            [pltpu.VMEM((SEQ, BPW), jnp.int32)]
            + [pltpu.VMEM((3 * HPW, EMBED), jnp.float32)] * 3
            + [pltpu.VMEM((BPW, EMBED), jnp.float32)]
            + [pltpu.SemaphoreType.DMA] * 3
        ),
    )(_w2v_body)
    return k(sentence, table)
